# baseline (speedup 1.0000x reference)
import functools

import jax
import jax.numpy as jnp
from jax import lax
from jax.experimental import pallas as pl
from jax.experimental.pallas import tpu as pltpu
from jax.experimental.pallas import tpu_sc as plsc

B, L, NC = 4096, 50, 26
V = 100000
ED = 32
ND = 13
DM = 128

TOK = B * L
SLOTS = 32
LINES = TOK * SLOTS * ED // 128
TOK_BLK = 128
HALF = 64
IDX_PER_STREAM = 128


def _idx_normalize(cat_t):
    def body(x_ref, out_ref):
        off = jax.lax.broadcasted_iota(jnp.int32, (NC, TOK_BLK), 0) * (V + 1)
        pad = jnp.full((SLOTS - NC, TOK_BLK), V, jnp.int32)
        for l in range(L):
            x32 = jnp.concatenate([x_ref[:, l, :] + off, pad], axis=0)
            y = (x32.reshape(8, 4, 8, 16)
                 .transpose(2, 1, 3, 0)
                 .reshape(SLOTS, TOK_BLK))
            out_ref[pl.ds(SLOTS * l, SLOTS), :] = y

    return pl.pallas_call(
        body,
        grid=(B // TOK_BLK,),
        in_specs=[pl.BlockSpec((NC, L, TOK_BLK), lambda i: (0, 0, i))],
        out_specs=pl.BlockSpec((L * SLOTS, TOK_BLK), lambda i: (i, 0)),
        out_shape=jax.ShapeDtypeStruct((B // TOK_BLK * L * SLOTS, TOK_BLK),
                                       jnp.int32),
    )(cat_t)


def _sc_gather(flat_table, c2):
    info = plsc.get_sparse_core_info()
    nw = info.num_cores * info.num_subcores
    total_sc = L * (B // TOK_BLK)
    assert total_sc % nw == 0
    sc_per_worker = total_sc // nw
    blocks_per_l = B // TOK_BLK

    mesh = plsc.VectorSubcoreMesh(core_axis_name="c", subcore_axis_name="s")

    @functools.partial(
        pl.kernel,
        out_type=jax.ShapeDtypeStruct((LINES, 128), jnp.float32),
        mesh=mesh,
        scratch_types=[
            pltpu.VMEM((SLOTS, TOK_BLK), jnp.int32),
            pltpu.VMEM((512, ED), jnp.float32),
            pltpu.VMEM((512, ED), jnp.float32),
            pltpu.VMEM((512, ED), jnp.float32),
            pltpu.VMEM((512, ED), jnp.float32),
            pltpu.SemaphoreType.DMA,
        ],
        compiler_params=pltpu.CompilerParams(use_tc_tiling_on_sc=False),
    )
    def gather_kernel(table_hbm, idx_hbm, out_hbm, s_idx,
                      buf0, buf1, buf2, buf3, sem):
        wid = lax.axis_index("s") * info.num_cores + lax.axis_index("c")
        bufs = (buf0, buf1, buf2, buf3)

        def do_half(t_base, h):
            cps = []
            for g in range(4):
                for a in range(4):
                    cps.append(pltpu.async_copy(
                        table_hbm.at[s_idx.at[16 * h + 4 * g + a]],
                        bufs[a].at[pl.ds(g * 128, 128)],
                        sem,
                    ))
            for cp in cps:
                cp.wait()
            line0 = pl.multiple_of((t_base + h * HALF) * 8, 8)
            for a in range(4):
                pltpu.sync_copy(
                    bufs[a],
                    out_hbm.at[pl.ds(line0, 512), pl.ds(a * ED, ED)])

        def sc_body(k, carry):
            scid = wid * sc_per_worker + k
            l = scid // blocks_per_l
            blk = scid % blocks_per_l
            row0 = pl.multiple_of((blk * L + l) * SLOTS, 8)
            pltpu.sync_copy(idx_hbm.at[pl.ds(row0, SLOTS)], s_idx)
            t_base = l * B + blk * TOK_BLK
            do_half(t_base, 0)
            do_half(t_base, 1)
            return carry

        lax.fori_loop(0, sc_per_worker, sc_body, 0, unroll=False)

    return gather_kernel(flat_table, c2)


def _tc_matmul_body(x_ref, num_ref, wp_ref, wn_ref, wf2_ref, bn_ref, bf_ref,
                    out_ref):
    acc = jnp.dot(x_ref[:, 0, :], wp_ref[pl.ds(0, 128), :],
                  preferred_element_type=jnp.float32)
    for r in range(1, 8):
        acc += jnp.dot(x_ref[:, r, :], wp_ref[pl.ds(r * 128, 128), :],
                       preferred_element_type=jnp.float32)
    l = pl.program_id(1)
    num_t = num_ref[:, pl.ds(l, 1), :][:, 0, :]
    num_proj = (
        jnp.dot(num_t.T, wn_ref[...], preferred_element_type=jnp.float32)
        + bn_ref[...]
    )
    acc += jnp.dot(num_proj, wf2_ref[...], preferred_element_type=jnp.float32)
    out_ref[...] = (acc + bf_ref[...])[None]


def _tc_matmul(x3, num_t, w_pad, w_num, wf_num, b_num, b_final):
    bb = 1024
    nb = B // bb
    return pl.pallas_call(
        _tc_matmul_body,
        grid=(nb, L),
        in_specs=[
            pl.BlockSpec((bb, 8, 128), lambda i, l: (l * nb + i, 0, 0)),
            pl.BlockSpec((ND, L, bb), lambda i, l: (0, 0, i)),
            pl.BlockSpec((SLOTS * ED, DM), lambda i, l: (0, 0)),
            pl.BlockSpec((ND, ED), lambda i, l: (0, 0)),
            pl.BlockSpec((ED, DM), lambda i, l: (0, 0)),
            pl.BlockSpec((1, ED), lambda i, l: (0, 0)),
            pl.BlockSpec((1, DM), lambda i, l: (0, 0)),
        ],
        out_specs=pl.BlockSpec((1, bb, DM), lambda i, l: (l, i, 0)),
        out_shape=jax.ShapeDtypeStruct((L, B, DM), jnp.float32),
    )(x3, num_t, w_pad, w_num, wf_num, b_num, b_final)


def kernel(cat_feats, num_feats, emb_tables, W_num, b_num, W_final, b_final):
    flat_table = emb_tables.reshape(NC * (V + 1), ED)
    c2 = _idx_normalize(cat_feats.astype(jnp.int32).transpose(2, 1, 0))

    x3 = _sc_gather(flat_table, c2).reshape(TOK, 8, 128)

    w_pad = jnp.concatenate(
        [W_final[: NC * ED],
         jnp.zeros((SLOTS * ED - NC * ED, DM), jnp.float32)])

    out3 = _tc_matmul(
        x3,
        num_feats.transpose(2, 1, 0),
        w_pad,
        W_num,
        W_final[NC * ED:],
        b_num.reshape(1, ED),
        b_final.reshape(1, DM),
    )
    return out3.transpose(1, 0, 2)


# speedup vs baseline: 1.7384x; 1.7384x over previous
import functools

import jax
import jax.numpy as jnp
from jax import lax
from jax.experimental import pallas as pl
from jax.experimental.pallas import tpu as pltpu
from jax.experimental.pallas import tpu_sc as plsc

B, L, NC = 4096, 50, 26
V = 100000
ED = 32
ND = 13
DM = 128

TOK = B * L
SLOTS = 32
IDX2 = TOK * SLOTS
VB = 2048
VP = 49 * VB
TROWS = NC * VP
IDX_PER_STREAM = 128
FIRES = 16
CHUNK = FIRES * IDX_PER_STREAM


def _table_lines(emb_t):
    def body(x_ref, out_ref):
        x = x_ref[0]
        xt3 = x.T.reshape(VB // 4, 4, ED)
        out_ref[0] = jnp.concatenate([xt3[:, a, :] for a in range(4)], axis=1)

    nvb = VP // VB

    return pl.pallas_call(
        body,
        grid=(NC, nvb),
        in_specs=[pl.BlockSpec((1, ED, VB), lambda i, j: (i, 0, j))],
        out_specs=pl.BlockSpec((1, VB // 4, 128), lambda i, j: (i, j, 0)),
        out_shape=jax.ShapeDtypeStruct((NC, VP // 4, 128), jnp.float32),
    )(emb_t)


def _sc_gather(flat_table, idx2d):
    info = plsc.get_sparse_core_info()
    nw = info.num_cores * info.num_subcores
    total_chunks = IDX2 // CHUNK
    assert total_chunks % nw == 0
    chunks_per_worker = total_chunks // nw

    mesh = plsc.VectorSubcoreMesh(core_axis_name="c", subcore_axis_name="s")

    @functools.partial(
        pl.kernel,
        out_type=jax.ShapeDtypeStruct((IDX2, ED), jnp.float32),
        mesh=mesh,
        scratch_types=[
            pltpu.VMEM((FIRES, IDX_PER_STREAM), jnp.int32),
            pltpu.VMEM((CHUNK, ED), jnp.float32),
            pltpu.SemaphoreType.DMA,
        ],
        compiler_params=pltpu.CompilerParams(use_tc_tiling_on_sc=False),
    )
    def gather_kernel(table_hbm, idx_hbm, out_hbm, idx_v, rows_v, sem):
        wid = lax.axis_index("s") * info.num_cores + lax.axis_index("c")

        def chunk_body(k, carry):
            cid = wid * chunks_per_worker + k
            idx_off = pl.multiple_of(cid * FIRES, 8)
            row_off = pl.multiple_of(cid * CHUNK, 8)
            pltpu.sync_copy(idx_hbm.at[pl.ds(idx_off, FIRES)], idx_v)
            cps = []
            for j in range(FIRES):
                cps.append(pltpu.async_copy(
                    table_hbm.at[idx_v.at[j]],
                    rows_v.at[pl.ds(j * IDX_PER_STREAM, IDX_PER_STREAM)],
                    sem,
                ))
            for cp in cps:
                cp.wait()
            pltpu.sync_copy(rows_v, out_hbm.at[pl.ds(row_off, CHUNK)])
            return carry

        lax.fori_loop(0, chunks_per_worker, chunk_body, 0, unroll=False)

    return gather_kernel(flat_table, idx2d)


def _tc_matmul_body(x_ref, num_ref, wp_ref, wn_ref, wf2_ref, bn_ref, bf_ref,
                    out_ref):
    acc = jnp.dot(x_ref[:, 0, :], wp_ref[pl.ds(0, 128), :],
                  preferred_element_type=jnp.float32)
    for r in range(1, 8):
        acc += jnp.dot(x_ref[:, r, :], wp_ref[pl.ds(r * 128, 128), :],
                       preferred_element_type=jnp.float32)
    num_proj = (
        jnp.dot(num_ref[...], wn_ref[...], preferred_element_type=jnp.float32)
        + bn_ref[...]
    )
    acc += jnp.dot(num_proj, wf2_ref[...], preferred_element_type=jnp.float32)
    out_ref[...] = acc + bf_ref[...]


def _tc_matmul(x3, num_flat, w_pad, w_num, wf_num, b_num, b_final):
    bm = 1024
    grid = (TOK // bm,)
    return pl.pallas_call(
        _tc_matmul_body,
        grid=grid,
        in_specs=[
            pl.BlockSpec((bm, 8, 128), lambda i: (i, 0, 0)),
            pl.BlockSpec((bm, ND), lambda i: (i, 0)),
            pl.BlockSpec((SLOTS * ED, DM), lambda i: (0, 0)),
            pl.BlockSpec((ND, ED), lambda i: (0, 0)),
            pl.BlockSpec((ED, DM), lambda i: (0, 0)),
            pl.BlockSpec((1, ED), lambda i: (0, 0)),
            pl.BlockSpec((1, DM), lambda i: (0, 0)),
        ],
        out_specs=pl.BlockSpec((bm, DM), lambda i: (i, 0)),
        out_shape=jax.ShapeDtypeStruct((TOK, DM), jnp.float32),
    )(x3, num_flat, w_pad, w_num, wf_num, b_num, b_final)


def kernel(cat_feats, num_feats, emb_tables, W_num, b_num, W_final, b_final):
    lines = _table_lines(emb_tables.transpose(0, 2, 1))
    flat_table = lines.reshape(TROWS, ED)

    offsets = (jnp.arange(NC, dtype=jnp.int32) * VP)[None, None, :]
    idx_real = cat_feats.astype(jnp.int32) + offsets
    idx_pad = jnp.full((B, L, SLOTS - NC), V, dtype=jnp.int32)
    idx2d = jnp.concatenate([idx_real, idx_pad], axis=-1).reshape(
        IDX2 // IDX_PER_STREAM, IDX_PER_STREAM)

    x3 = _sc_gather(flat_table, idx2d).reshape(TOK, 8, 128)

    w_pad = jnp.concatenate(
        [W_final[: NC * ED],
         jnp.zeros((SLOTS * ED - NC * ED, DM), jnp.float32)])

    out = _tc_matmul(
        x3,
        num_feats.reshape(TOK, ND),
        w_pad,
        W_num,
        W_final[NC * ED:],
        b_num.reshape(1, ED),
        b_final.reshape(1, DM),
    )
    return out.reshape(B, L, DM)


# speedup vs baseline: 9.5839x; 5.5131x over previous
import functools

import jax
import jax.numpy as jnp
from jax import lax
from jax.experimental import pallas as pl
from jax.experimental.pallas import tpu as pltpu
from jax.experimental.pallas import tpu_sc as plsc

B, L, NC = 4096, 50, 26
V = 100000
ED = 32
ND = 13
DM = 128

TOK = B * L
SLOTS = 32
IDX2 = TOK * SLOTS
VB = 2048
VP = 49 * VB
TROWS = NC * VP
IDX_PER_STREAM = 128
FIRES = 16
CHUNK = FIRES * IDX_PER_STREAM


def _table_lines(emb_t):
    def body(x_ref, out_ref):
        j = pl.program_id(1)
        x = x_ref[0]
        vid = lax.broadcasted_iota(jnp.int32, (ED, VB), 1) + j * VB
        x = jnp.where(vid <= V, x, 0.0)
        xt3 = x.T.reshape(VB // 4, 4, ED)
        out_ref[0] = jnp.concatenate([xt3[:, a, :] for a in range(4)], axis=1)

    nvb = VP // VB

    return pl.pallas_call(
        body,
        grid=(NC, nvb),
        in_specs=[pl.BlockSpec((1, ED, VB), lambda i, j: (i, 0, j))],
        out_specs=pl.BlockSpec((1, VB // 4, 128), lambda i, j: (i, j, 0)),
        out_shape=jax.ShapeDtypeStruct((NC, VP // 4, 128), jnp.float32),
    )(emb_t)


def _sc_gather(flat_table, idx2d):
    info = plsc.get_sparse_core_info()
    nw = info.num_cores * info.num_subcores
    total_chunks = IDX2 // CHUNK
    assert total_chunks % nw == 0
    chunks_per_worker = total_chunks // nw

    mesh = plsc.VectorSubcoreMesh(core_axis_name="c", subcore_axis_name="s")

    @functools.partial(
        pl.kernel,
        out_type=jax.ShapeDtypeStruct((IDX2, ED), jnp.float32),
        mesh=mesh,
        scratch_types=[
            pltpu.VMEM((FIRES, IDX_PER_STREAM), jnp.int32),
            pltpu.VMEM((CHUNK, ED), jnp.float32),
            pltpu.SemaphoreType.DMA,
        ],
        compiler_params=pltpu.CompilerParams(use_tc_tiling_on_sc=False),
    )
    def gather_kernel(table_hbm, idx_hbm, out_hbm, idx_v, rows_v, sem):
        wid = lax.axis_index("s") * info.num_cores + lax.axis_index("c")

        def chunk_body(k, carry):
            cid = wid * chunks_per_worker + k
            idx_off = pl.multiple_of(cid * FIRES, 8)
            row_off = pl.multiple_of(cid * CHUNK, 8)
            pltpu.sync_copy(idx_hbm.at[pl.ds(idx_off, FIRES)], idx_v)
            cps = []
            for j in range(FIRES):
                cps.append(pltpu.async_copy(
                    table_hbm.at[idx_v.at[j]],
                    rows_v.at[pl.ds(j * IDX_PER_STREAM, IDX_PER_STREAM)],
                    sem,
                ))
            for cp in cps:
                cp.wait()
            pltpu.sync_copy(rows_v, out_hbm.at[pl.ds(row_off, CHUNK)])
            return carry

        lax.fori_loop(0, chunks_per_worker, chunk_body, 0, unroll=False)

    return gather_kernel(flat_table, idx2d)


def _tc_matmul_body(x_ref, num_ref, wp_ref, wn_ref, wf2_ref, bn_ref, bf_ref,
                    out_ref):
    acc = jnp.dot(x_ref[:, 0, :], wp_ref[pl.ds(0, 128), :],
                  preferred_element_type=jnp.float32)
    for r in range(1, 8):
        acc += jnp.dot(x_ref[:, r, :], wp_ref[pl.ds(r * 128, 128), :],
                       preferred_element_type=jnp.float32)
    num_proj = (
        jnp.dot(num_ref[...], wn_ref[...], preferred_element_type=jnp.float32)
        + bn_ref[...]
    )
    acc += jnp.dot(num_proj, wf2_ref[...], preferred_element_type=jnp.float32)
    out_ref[...] = acc + bf_ref[...]


def _tc_matmul(x3, num_flat, w_pad, w_num, wf_num, b_num, b_final):
    bm = 1024
    grid = (TOK // bm,)
    return pl.pallas_call(
        _tc_matmul_body,
        grid=grid,
        in_specs=[
            pl.BlockSpec((bm, 8, 128), lambda i: (i, 0, 0)),
            pl.BlockSpec((bm, ND), lambda i: (i, 0)),
            pl.BlockSpec((SLOTS * ED, DM), lambda i: (0, 0)),
            pl.BlockSpec((ND, ED), lambda i: (0, 0)),
            pl.BlockSpec((ED, DM), lambda i: (0, 0)),
            pl.BlockSpec((1, ED), lambda i: (0, 0)),
            pl.BlockSpec((1, DM), lambda i: (0, 0)),
        ],
        out_specs=pl.BlockSpec((bm, DM), lambda i: (i, 0)),
        out_shape=jax.ShapeDtypeStruct((TOK, DM), jnp.float32),
    )(x3, num_flat, w_pad, w_num, wf_num, b_num, b_final)


def kernel(cat_feats, num_feats, emb_tables, W_num, b_num, W_final, b_final):
    lines = _table_lines(emb_tables.transpose(0, 2, 1))
    flat_table = lines.reshape(TROWS, ED)

    offsets = (jnp.arange(NC, dtype=jnp.int32) * VP)[None, None, :]
    idx_real = cat_feats.astype(jnp.int32) + offsets
    npad = SLOTS - NC
    spread = NC * (VP - V - 1)
    p = (lax.broadcasted_iota(jnp.int32, (B, L, npad), 0) * (L * npad)
         + lax.broadcasted_iota(jnp.int32, (B, L, npad), 1) * npad
         + lax.broadcasted_iota(jnp.int32, (B, L, npad), 2))
    q = p % spread
    idx_pad = (q // (VP - V - 1)) * VP + (V + 1) + q % (VP - V - 1)
    idx2d = jnp.concatenate([idx_real, idx_pad], axis=-1).reshape(
        IDX2 // IDX_PER_STREAM, IDX_PER_STREAM)

    x3 = _sc_gather(flat_table, idx2d).reshape(TOK, 8, 128)

    w_pad = jnp.concatenate(
        [W_final[: NC * ED],
         jnp.zeros((SLOTS * ED - NC * ED, DM), jnp.float32)])

    out = _tc_matmul(
        x3,
        num_feats.reshape(TOK, ND),
        w_pad,
        W_num,
        W_final[NC * ED:],
        b_num.reshape(1, ED),
        b_final.reshape(1, DM),
    )
    return out.reshape(B, L, DM)


# speedup vs baseline: 10.2140x; 1.0657x over previous
import functools

import jax
import jax.numpy as jnp
from jax import lax
from jax.experimental import pallas as pl
from jax.experimental.pallas import tpu as pltpu
from jax.experimental.pallas import tpu_sc as plsc

B, L, NC = 4096, 50, 26
V = 100000
ED = 32
ND = 13
DM = 128

TOK = B * L
SLOTS = 32
IDX2 = TOK * SLOTS
VB = 4096
VP = 25 * VB
TROWS = NC * VP
IDX_PER_STREAM = 128
FIRES = 16
CHUNK = FIRES * IDX_PER_STREAM


def _table_lines(emb_t):
    def body(x_ref, out_ref):
        j = pl.program_id(1)
        x = x_ref[0]
        vid = lax.broadcasted_iota(jnp.int32, (ED, VB), 1) + j * VB
        x = jnp.where(vid <= V, x, 0.0)
        xt3 = x.T.reshape(VB // 4, 4, ED)
        out_ref[0] = jnp.concatenate([xt3[:, a, :] for a in range(4)], axis=1)

    nvb = VP // VB

    return pl.pallas_call(
        body,
        grid=(NC, nvb),
        in_specs=[pl.BlockSpec((1, ED, VB), lambda i, j: (i, 0, j))],
        out_specs=pl.BlockSpec((1, VB // 4, 128), lambda i, j: (i, j, 0)),
        out_shape=jax.ShapeDtypeStruct((NC, VP // 4, 128), jnp.float32),
    )(emb_t)


def _sc_gather(flat_table, idx2d):
    info = plsc.get_sparse_core_info()
    nw = info.num_cores * info.num_subcores
    total_chunks = IDX2 // CHUNK
    assert total_chunks % nw == 0
    chunks_per_worker = total_chunks // nw

    mesh = plsc.VectorSubcoreMesh(core_axis_name="c", subcore_axis_name="s")

    @functools.partial(
        pl.kernel,
        out_type=jax.ShapeDtypeStruct((IDX2, ED), jnp.float32),
        mesh=mesh,
        scratch_types=[
            pltpu.VMEM((FIRES, IDX_PER_STREAM), jnp.int32),
            pltpu.VMEM((CHUNK, ED), jnp.float32),
            pltpu.SemaphoreType.DMA,
        ],
        compiler_params=pltpu.CompilerParams(use_tc_tiling_on_sc=False),
    )
    def gather_kernel(table_hbm, idx_hbm, out_hbm, idx_v, rows_v, sem):
        wid = lax.axis_index("s") * info.num_cores + lax.axis_index("c")

        def chunk_body(k, carry):
            cid = wid * chunks_per_worker + k
            idx_off = pl.multiple_of(cid * FIRES, 8)
            row_off = pl.multiple_of(cid * CHUNK, 8)
            pltpu.sync_copy(idx_hbm.at[pl.ds(idx_off, FIRES)], idx_v)
            cps = []
            for j in range(FIRES):
                cps.append(pltpu.async_copy(
                    table_hbm.at[idx_v.at[j]],
                    rows_v.at[pl.ds(j * IDX_PER_STREAM, IDX_PER_STREAM)],
                    sem,
                ))
            for cp in cps:
                cp.wait()
            pltpu.sync_copy(rows_v, out_hbm.at[pl.ds(row_off, CHUNK)])
            return carry

        lax.fori_loop(0, chunks_per_worker, chunk_body, 0, unroll=False)

    return gather_kernel(flat_table, idx2d)


def _tc_matmul_body(x_ref, num_ref, wp_ref, wn_ref, wf2_ref, bn_ref, bf_ref,
                    out_ref):
    acc = jnp.dot(x_ref[:, 0, :], wp_ref[pl.ds(0, 128), :],
                  preferred_element_type=jnp.float32)
    for r in range(1, 8):
        acc += jnp.dot(x_ref[:, r, :], wp_ref[pl.ds(r * 128, 128), :],
                       preferred_element_type=jnp.float32)
    num_proj = (
        jnp.dot(num_ref[...], wn_ref[...], preferred_element_type=jnp.float32)
        + bn_ref[...]
    )
    acc += jnp.dot(num_proj, wf2_ref[...], preferred_element_type=jnp.float32)
    out_ref[...] = acc + bf_ref[...]


def _tc_matmul(x3, num_flat, w_pad, w_num, wf_num, b_num, b_final):
    bm = 1024
    grid = (TOK // bm,)
    return pl.pallas_call(
        _tc_matmul_body,
        grid=grid,
        in_specs=[
            pl.BlockSpec((bm, 8, 128), lambda i: (i, 0, 0)),
            pl.BlockSpec((bm, ND), lambda i: (i, 0)),
            pl.BlockSpec((SLOTS * ED, DM), lambda i: (0, 0)),
            pl.BlockSpec((ND, ED), lambda i: (0, 0)),
            pl.BlockSpec((ED, DM), lambda i: (0, 0)),
            pl.BlockSpec((1, ED), lambda i: (0, 0)),
            pl.BlockSpec((1, DM), lambda i: (0, 0)),
        ],
        out_specs=pl.BlockSpec((bm, DM), lambda i: (i, 0)),
        out_shape=jax.ShapeDtypeStruct((TOK, DM), jnp.float32),
    )(x3, num_flat, w_pad, w_num, wf_num, b_num, b_final)


def kernel(cat_feats, num_feats, emb_tables, W_num, b_num, W_final, b_final):
    lines = _table_lines(emb_tables.transpose(0, 2, 1))
    flat_table = lines.reshape(TROWS, ED)

    offsets = (jnp.arange(NC, dtype=jnp.int32) * VP)[None, None, :]
    idx_real = cat_feats.astype(jnp.int32) + offsets
    npad = SLOTS - NC
    spread = NC * (VP - V - 1)
    p = (lax.broadcasted_iota(jnp.int32, (B, L, npad), 0) * (L * npad)
         + lax.broadcasted_iota(jnp.int32, (B, L, npad), 1) * npad
         + lax.broadcasted_iota(jnp.int32, (B, L, npad), 2))
    q = p % spread
    idx_pad = (q // (VP - V - 1)) * VP + (V + 1) + q % (VP - V - 1)
    idx2d = jnp.concatenate([idx_real, idx_pad], axis=-1).reshape(
        IDX2 // IDX_PER_STREAM, IDX_PER_STREAM)

    x3 = _sc_gather(flat_table, idx2d).reshape(TOK, 8, 128)

    w_pad = jnp.concatenate(
        [W_final[: NC * ED],
         jnp.zeros((SLOTS * ED - NC * ED, DM), jnp.float32)])

    out = _tc_matmul(
        x3,
        num_feats.reshape(TOK, ND),
        w_pad,
        W_num,
        W_final[NC * ED:],
        b_num.reshape(1, ED),
        b_final.reshape(1, DM),
    )
    return out.reshape(B, L, DM)
